# baseline (speedup 1.0000x reference)
import jax
import jax.numpy as jnp
from jax import lax
from jax.experimental import pallas as pl
from jax.experimental.pallas import tpu as pltpu
from jax.experimental.pallas import tpu_sc as plsc

_NC, _NS = 2, 16
_PAD_ROWS = 8000
_LANES = 16


def _pad_table(table):
    v, d = table.shape

    def body(t_ref, o_ref):
        o_ref[:, :d] = t_ref[...]

    return pl.pallas_call(
        body,
        grid=(v // _PAD_ROWS,),
        in_specs=[pl.BlockSpec((_PAD_ROWS, d), lambda i: (i, 0))],
        out_specs=pl.BlockSpec((_PAD_ROWS, 128), lambda i: (i, 0)),
        out_shape=jax.ShapeDtypeStruct((v, 128), table.dtype),
    )(table)


def kernel(inputs, table):
    b, s = inputs.shape
    n = b * s
    v, d = table.shape
    nw = _NC * _NS
    per_w = n // nw
    n_chunks = per_w // s
    n_pairs = n_chunks // 2
    idx = inputs.reshape(n).astype(jnp.int32)
    table_pad = _pad_table(table)
    mesh = plsc.VectorSubcoreMesh(core_axis_name="c", subcore_axis_name="s")

    @pl.kernel(
        out_type=jax.ShapeDtypeStruct((b, s, d), table.dtype),
        mesh=mesh,
        scratch_types=[
            pltpu.VMEM((s,), jnp.int32),
            pltpu.VMEM((s,), jnp.int32),
            pltpu.VMEM((s, 128), jnp.float32),
            pltpu.VMEM((s, 128), jnp.float32),
            pltpu.VMEM((s, d), jnp.float32),
            pltpu.VMEM((s, d), jnp.float32),
            pltpu.SemaphoreType.DMA,
            pltpu.SemaphoreType.DMA,
            pltpu.SemaphoreType.DMA,
            pltpu.SemaphoreType.DMA,
        ],
    )
    def gather_kernel(table_hbm, idx_hbm, out_hbm,
                      i0, i1, r0, r1, o0, o1, g0, g1, w0, w1):
        wid = lax.axis_index("s") * _NC + lax.axis_index("c")
        base = wid * per_w
        seq0 = wid * n_chunks
        ibufs, rbufs, obufs = (i0, i1), (r0, r1), (o0, o1)
        gsems, wsems = (g0, g1), (w0, w1)

        def compact(p):
            r_ref, o_ref = rbufs[p], obufs[p]

            @plsc.parallel_loop(0, s)
            def _(j):
                for q in range(d // _LANES):
                    o_ref[j, pl.ds(q * _LANES, _LANES)] = (
                        r_ref[j, pl.ds(q * _LANES, _LANES)])

        def wait_gather(p):
            pltpu.make_async_copy(table_hbm.at[ibufs[p]], rbufs[p],
                                  gsems[p]).wait()

        def wait_write(p, seq):
            pltpu.make_async_copy(obufs[p], out_hbm.at[seq], wsems[p]).wait()

        for p in range(2):
            pltpu.sync_copy(idx_hbm.at[pl.ds(base + p * s, s)], ibufs[p])
            pltpu.async_copy(table_hbm.at[ibufs[p]], rbufs[p], gsems[p])

        for p in range(2):
            wait_gather(p)
            compact(p)
            pltpu.async_copy(obufs[p], out_hbm.at[seq0 + p], wsems[p])
            pltpu.sync_copy(idx_hbm.at[pl.ds(base + (p + 2) * s, s)],
                            ibufs[p])
            pltpu.async_copy(table_hbm.at[ibufs[p]], rbufs[p], gsems[p])

        @pl.loop(1, n_pairs - 1)
        def _(pair):
            for p in range(2):
                c = 2 * pair + p
                wait_gather(p)
                wait_write(p, seq0 + c - 2)
                compact(p)
                pltpu.async_copy(obufs[p], out_hbm.at[seq0 + c], wsems[p])
                pltpu.sync_copy(
                    idx_hbm.at[pl.ds(base + (c + 2) * s, s)], ibufs[p])
                pltpu.async_copy(table_hbm.at[ibufs[p]], rbufs[p], gsems[p])

        for p in range(2):
            c = n_chunks - 2 + p
            wait_gather(p)
            wait_write(p, seq0 + c - 2)
            compact(p)
            pltpu.async_copy(obufs[p], out_hbm.at[seq0 + c], wsems[p])
        for p in range(2):
            wait_write(p, seq0 + n_chunks - 2 + p)

    return gather_kernel(table_pad, idx)


# speedup vs baseline: 1.1663x; 1.1663x over previous
import jax
import jax.numpy as jnp
from jax import lax
from jax.experimental import pallas as pl
from jax.experimental.pallas import tpu as pltpu
from jax.experimental.pallas import tpu_sc as plsc

_NC, _NS = 2, 16
_CHUNK = 400
_GROUP = 16
_PAD_ROWS = 8000


def _pad_table(table):
    v, d = table.shape

    def body(t_ref, o_ref):
        o_ref[:, :d] = t_ref[...]

    return pl.pallas_call(
        body,
        grid=(v // _PAD_ROWS,),
        in_specs=[pl.BlockSpec((_PAD_ROWS, d), lambda i: (i, 0))],
        out_specs=pl.BlockSpec((_PAD_ROWS, 128), lambda i: (i, 0)),
        out_shape=jax.ShapeDtypeStruct((v, 128), table.dtype),
    )(table)


def kernel(inputs, table):
    b, s = inputs.shape
    n = b * s
    v, d = table.shape
    nw = _NC * _NS
    per_w = n // nw
    n_groups = per_w // (_GROUP * _CHUNK)
    idx = inputs.reshape(n).astype(jnp.int32)
    table_pad = _pad_table(table)
    mesh = plsc.VectorSubcoreMesh(core_axis_name="c", subcore_axis_name="s")

    @pl.kernel(
        out_type=jax.ShapeDtypeStruct((n, 128), table.dtype),
        mesh=mesh,
        scratch_types=[
            pltpu.VMEM((_GROUP * _CHUNK,), jnp.int32),
            pltpu.VMEM((_CHUNK, 128), jnp.float32),
            pltpu.VMEM((_CHUNK, 128), jnp.float32),
            pltpu.SemaphoreType.DMA,
            pltpu.SemaphoreType.DMA,
            pltpu.SemaphoreType.DMA,
            pltpu.SemaphoreType.DMA,
        ],
    )
    def gather_kernel(table_hbm, idx_hbm, out_hbm,
                      ibuf, r0, r1, g0, g1, w0, w1):
        wid = lax.axis_index("s") * _NC + lax.axis_index("c")
        base = wid * per_w
        rbufs, gsems, wsems = (r0, r1), (g0, g1), (w0, w1)

        def fire_gather(p, k):
            pltpu.async_copy(
                table_hbm.at[ibuf.at[pl.ds(k * _CHUNK, _CHUNK)]],
                rbufs[p], gsems[p])

        def wait_gather(p, k):
            pltpu.make_async_copy(
                table_hbm.at[ibuf.at[pl.ds(k * _CHUNK, _CHUNK)]],
                rbufs[p], gsems[p]).wait()

        @pl.loop(0, n_groups)
        def _(g):
            goff = base + g * (_GROUP * _CHUNK)
            pltpu.sync_copy(idx_hbm.at[pl.ds(goff, _GROUP * _CHUNK)], ibuf)
            for p in range(2):
                fire_gather(p, p)

            @pl.loop(0, _GROUP // 2 - 1)
            def _(j):
                for p in range(2):
                    k = 2 * j + p
                    woff = goff + k * _CHUNK
                    wait_gather(p, k)
                    pltpu.async_copy(rbufs[p],
                                     out_hbm.at[pl.ds(woff, _CHUNK)],
                                     wsems[p])
                    pltpu.make_async_copy(rbufs[p],
                                          out_hbm.at[pl.ds(woff, _CHUNK)],
                                          wsems[p]).wait()
                    fire_gather(p, k + 2)

            for p in range(2):
                k = _GROUP - 2 + p
                wait_gather(p, k)
                pltpu.async_copy(rbufs[p],
                                 out_hbm.at[pl.ds(goff + k * _CHUNK, _CHUNK)],
                                 wsems[p])
            for p in range(2):
                k = _GROUP - 2 + p
                pltpu.make_async_copy(rbufs[p],
                                      out_hbm.at[pl.ds(goff + k * _CHUNK,
                                                       _CHUNK)],
                                      wsems[p]).wait()

    out = gather_kernel(table_pad, idx)
    return out[:, :d].reshape(b, s, d)
